# baseline (speedup 1.0000x reference)
import functools

import jax
import jax.numpy as jnp
from jax import lax
from jax.experimental import pallas as pl
from jax.experimental.pallas import tpu as pltpu
from jax.experimental.pallas import tpu_sc as plsc

N = 10000
E = 320000
D = 128
H1 = 64
H2 = 32

NC = 2
NS = 16
NW = NC * NS

EPT = E // NW
OCH = 5
CW = 80
EPT_PAD = 10240
DCH = EPT // OCH
NP = 10240
ROWS_PT = NP // NS
ZR = 128

_mesh = lambda: plsc.VectorSubcoreMesh(core_axis_name="c", subcore_axis_name="s")


@functools.partial(
    pl.kernel,
    out_type=jax.ShapeDtypeStruct((NW, N), jnp.float32),
    mesh=_mesh(),
    scratch_types=[
        pltpu.VMEM((DCH,), jnp.int32),
        pltpu.VMEM((N,), jnp.float32),
    ],
    compiler_params=pltpu.CompilerParams(needs_layout_passes=False),
)
def _deg_kernel(dst_hbm, out_hbm, dstv, degv):
    cid = lax.axis_index("c")
    sid = lax.axis_index("s")
    wid = cid * NS + sid

    zero16 = jnp.zeros((16,), jnp.float32)

    def zbody(i, carry):
        degv[pl.ds(i * 16, 16)] = zero16
        return carry

    lax.fori_loop(0, N // 16, zbody, 0)

    ones16 = jnp.ones((16,), jnp.float32)

    def outer(o, carry):
        pltpu.sync_copy(dst_hbm.at[pl.ds(wid * EPT + o * DCH, DCH)], dstv)

        def inner(j, c2):
            idx = dstv[pl.ds(j * 16, 16)]
            plsc.addupdate_scatter(degv, [idx], ones16)
            return c2

        lax.fori_loop(0, DCH // 16, inner, 0)
        return carry

    lax.fori_loop(0, OCH, outer, 0)
    pltpu.sync_copy(degv, out_hbm.at[wid])


NCH = EPT_PAD // CW
NBUF = 4


def _make_prop(H):
    @functools.partial(
        pl.kernel,
        out_type=jax.ShapeDtypeStruct((NC, NP, H), jnp.float32),
        mesh=_mesh(),
        scratch_types=[
            pltpu.VMEM((NCH, CW), jnp.int32),
            pltpu.VMEM((NCH, CW), jnp.int32),
            pltpu.VMEM((NBUF, CW, H), jnp.float32),
            pltpu.VMEM_SHARED((NP, H), jnp.float32),
            pltpu.SemaphoreType.DMA((2,)),
            pltpu.SemaphoreType.DMA((NBUF,)),
            pltpu.SemaphoreType.DMA((NBUF,)),
        ],
        compiler_params=pltpu.CompilerParams(
            needs_layout_passes=False, use_tc_tiling_on_sc=False
        ),
    )
    def _prop(h_hbm, src_hbm, dst_hbm, out_hbm, src_all, dst_all, rows,
              acc, sem_i, sem_g, sem_s):
        cid = lax.axis_index("c")
        sid = lax.axis_index("s")
        wid = cid * NS + sid

        cp_src = pltpu.async_copy(src_hbm.at[wid], src_all, sem_i.at[0])
        cp_dst = pltpu.async_copy(dst_hbm.at[wid], dst_all, sem_i.at[1])

        zero16 = jnp.zeros((16,), jnp.float32)

        def zbody(i, carry):
            for c in range(H // 16):
                rows[0, i, pl.ds(c * 16, 16)] = zero16
            return carry

        lax.fori_loop(0, CW, zbody, 0)
        for k in range(ROWS_PT // CW):
            pltpu.sync_copy(rows.at[0], acc.at[pl.ds(sid * ROWS_PT + k * CW, CW)])
        cp_src.wait()
        cp_dst.wait()
        plsc.subcore_barrier()

        def gather_wait(i, b):
            pltpu.make_async_copy(
                h_hbm.at[src_all.at[i]], rows.at[b], sem_g.at[b]
            ).wait()

        def gather_start(i, b):
            pltpu.async_copy(h_hbm.at[src_all.at[i]], rows.at[b], sem_g.at[b])

        def scatter_wait(b):
            pltpu.make_async_copy(
                rows.at[b], acc.at[pl.ds(0, CW)], sem_s.at[b]
            ).wait()

        def scatter_start(i, b):
            pltpu.async_copy(
                rows.at[b], acc.at[dst_all.at[i]], sem_s.at[b], add=True
            )

        gather_start(0, 0)
        for b in range(NBUF):
            gather_wait(b, b)
            scatter_start(b, b)
            if b == NBUF - 1:
                scatter_wait(0)
            gather_start(b + 1, (b + 1) % NBUF)

        def outer(o, carry):
            for b in range(NBUF):
                i = o * NBUF + b
                gather_wait(i, b)
                scatter_start(i, b)
                scatter_wait((b + 1) % NBUF)
                gather_start(i + 1, (b + 1) % NBUF)
            return carry

        lax.fori_loop(1, NCH // NBUF - 1, outer, 0)

        for b in range(NBUF):
            i = NCH - NBUF + b
            gather_wait(i, b)
            scatter_start(i, b)
            if b < NBUF - 1:
                scatter_wait((b + 1) % NBUF)
                gather_start(i + 1, (b + 1) % NBUF)
        for b in range(NBUF):
            scatter_wait(b)

        plsc.subcore_barrier()
        pltpu.sync_copy(
            acc.at[pl.ds(sid * ROWS_PT, ROWS_PT)],
            out_hbm.at[cid, pl.ds(sid * ROWS_PT, ROWS_PT)],
        )

    return _prop


_prop_h1 = _make_prop(H1)
_prop_h2 = _make_prop(H2)


def _inv_sqrt_deg(degp_blk):
    deg = jnp.sum(degp_blk, axis=0)
    return lax.rsqrt(jnp.maximum(deg, 1.0))


def _tc1_body(degp_ref, feat_ref, w1_ref, out_ref):
    a = _inv_sqrt_deg(degp_ref[...])
    y = jnp.dot(feat_ref[...], w1_ref[...], preferred_element_type=jnp.float32)
    out_ref[...] = y * a[:, None]


def _tc2_body(p_ref, degp_ref, w2_ref, out_ref):
    a = _inv_sqrt_deg(degp_ref[...])
    s = (p_ref[0, :N] + p_ref[1, :N]) * a[:, None]
    h1 = jnp.maximum(s, 0.0)
    y = jnp.dot(h1, w2_ref[...], preferred_element_type=jnp.float32)
    out_ref[...] = y * a[:, None]


def _tc3_body(p_ref, degp_ref, out_ref):
    a = _inv_sqrt_deg(degp_ref[...])
    out_ref[...] = (p_ref[0, :N] + p_ref[1, :N]) * a[:, None]


def _tc1(degp, feat, W1):
    return pl.pallas_call(
        _tc1_body,
        out_shape=jax.ShapeDtypeStruct((N, H1), jnp.float32),
    )(degp, feat, W1)


def _tc2(p1, degp, W2):
    return pl.pallas_call(
        _tc2_body,
        out_shape=jax.ShapeDtypeStruct((N, H2), jnp.float32),
    )(p1, degp, W2)


def _tc3(p2, degp):
    return pl.pallas_call(
        _tc3_body,
        out_shape=jax.ShapeDtypeStruct((N, H2), jnp.float32),
    )(p2, degp)


def kernel(feat, edge_index, W1, W2):
    src = jnp.concatenate(
        [edge_index[0].reshape(NW, EPT),
         jnp.zeros((NW, EPT_PAD - EPT), jnp.int32)], axis=1
    ).reshape(NW, NCH, CW)
    pad_dst = jnp.broadcast_to(
        N + jnp.arange(EPT_PAD - EPT, dtype=jnp.int32) % (NP - N),
        (NW, EPT_PAD - EPT),
    )
    dst = jnp.concatenate(
        [edge_index[1].reshape(NW, EPT), pad_dst], axis=1
    ).reshape(NW, NCH, CW)
    dst_flat = edge_index[1]

    degp = _deg_kernel(dst_flat)
    g1 = _tc1(degp, feat, W1)
    p1 = _prop_h1(g1, src, dst)
    g2 = _tc2(p1, degp, W2)
    p2 = _prop_h2(g2, src, dst)
    return _tc3(p2, degp)


# speedup vs baseline: 2.0296x; 2.0296x over previous
import functools

import jax
import jax.numpy as jnp
from jax import lax
from jax.experimental import pallas as pl
from jax.experimental.pallas import tpu as pltpu
from jax.experimental.pallas import tpu_sc as plsc

N = 10000
E = 320000
D = 128
H1 = 64
H2 = 32

NC = 2
NS = 16
NW = NC * NS

EPT = E // NW
OCH = 5
CW = 128
EPT_PAD = 10240
DCH = EPT // OCH
NP = 10240
ROWS_PT = NP // NS
ZR = 128

_mesh = lambda: plsc.VectorSubcoreMesh(core_axis_name="c", subcore_axis_name="s")


@functools.partial(
    pl.kernel,
    out_type=jax.ShapeDtypeStruct((NW, N), jnp.float32),
    mesh=_mesh(),
    scratch_types=[
        pltpu.VMEM((DCH,), jnp.int32),
        pltpu.VMEM((N,), jnp.float32),
    ],
    compiler_params=pltpu.CompilerParams(needs_layout_passes=False),
)
def _deg_kernel(dst_hbm, out_hbm, dstv, degv):
    cid = lax.axis_index("c")
    sid = lax.axis_index("s")
    wid = cid * NS + sid

    zero16 = jnp.zeros((16,), jnp.float32)

    def zbody(i, carry):
        degv[pl.ds(i * 16, 16)] = zero16
        return carry

    lax.fori_loop(0, N // 16, zbody, 0)

    ones16 = jnp.ones((16,), jnp.float32)

    def outer(o, carry):
        pltpu.sync_copy(dst_hbm.at[pl.ds(wid * EPT + o * DCH, DCH)], dstv)

        def inner(j, c2):
            idx = dstv[pl.ds(j * 16, 16)]
            plsc.addupdate_scatter(degv, [idx], ones16)
            return c2

        lax.fori_loop(0, DCH // 16, inner, 0)
        return carry

    lax.fori_loop(0, OCH, outer, 0)
    pltpu.sync_copy(degv, out_hbm.at[wid])


NCH = EPT_PAD // CW
NBUF = 8


def _make_prop(H):
    @functools.partial(
        pl.kernel,
        out_type=jax.ShapeDtypeStruct((NC, NP, H), jnp.float32),
        mesh=_mesh(),
        scratch_types=[
            pltpu.VMEM((NCH, CW), jnp.int32),
            pltpu.VMEM((NCH, CW), jnp.int32),
            pltpu.VMEM((NBUF, CW, H), jnp.float32),
            pltpu.VMEM_SHARED((NP, H), jnp.float32),
            pltpu.SemaphoreType.DMA((2,)),
            pltpu.SemaphoreType.DMA((NBUF,)),
            pltpu.SemaphoreType.DMA((NBUF,)),
        ],
        compiler_params=pltpu.CompilerParams(
            needs_layout_passes=False, use_tc_tiling_on_sc=False
        ),
    )
    def _prop(h_hbm, src_hbm, dst_hbm, out_hbm, src_all, dst_all, rows,
              acc, sem_i, sem_g, sem_s):
        cid = lax.axis_index("c")
        sid = lax.axis_index("s")
        wid = cid * NS + sid

        cp_src = pltpu.async_copy(src_hbm.at[wid], src_all, sem_i.at[0])
        cp_dst = pltpu.async_copy(dst_hbm.at[wid], dst_all, sem_i.at[1])

        zero16 = jnp.zeros((16,), jnp.float32)

        def zbody(i, carry):
            for c in range(H // 16):
                rows[0, i, pl.ds(c * 16, 16)] = zero16
            return carry

        lax.fori_loop(0, CW, zbody, 0)
        for k in range(ROWS_PT // CW):
            pltpu.sync_copy(rows.at[0], acc.at[pl.ds(sid * ROWS_PT + k * CW, CW)])
        cp_src.wait()
        cp_dst.wait()
        plsc.subcore_barrier()

        def gather_wait(i, b):
            pltpu.make_async_copy(
                h_hbm.at[src_all.at[i]], rows.at[b], sem_g.at[b]
            ).wait()

        def gather_start(i, b):
            pltpu.async_copy(h_hbm.at[src_all.at[i]], rows.at[b], sem_g.at[b])

        def scatter_wait(b):
            pltpu.make_async_copy(
                rows.at[b], acc.at[pl.ds(0, CW)], sem_s.at[b]
            ).wait()

        def scatter_start(i, b):
            pltpu.async_copy(
                rows.at[b], acc.at[dst_all.at[i]], sem_s.at[b], add=True
            )

        gather_start(0, 0)
        for b in range(NBUF):
            gather_wait(b, b)
            scatter_start(b, b)
            if b == NBUF - 1:
                scatter_wait(0)
            gather_start(b + 1, (b + 1) % NBUF)

        def outer(o, carry):
            for b in range(NBUF):
                i = o * NBUF + b
                gather_wait(i, b)
                scatter_start(i, b)
                scatter_wait((b + 1) % NBUF)
                gather_start(i + 1, (b + 1) % NBUF)
            return carry

        lax.fori_loop(1, NCH // NBUF - 1, outer, 0)

        for b in range(NBUF):
            i = NCH - NBUF + b
            gather_wait(i, b)
            scatter_start(i, b)
            if b < NBUF - 1:
                scatter_wait((b + 1) % NBUF)
                gather_start(i + 1, (b + 1) % NBUF)
        for b in range(NBUF):
            scatter_wait(b)

        plsc.subcore_barrier()
        pltpu.sync_copy(
            acc.at[pl.ds(sid * ROWS_PT, ROWS_PT)],
            out_hbm.at[cid, pl.ds(sid * ROWS_PT, ROWS_PT)],
        )

    return _prop


_prop_h1 = _make_prop(H1)
_prop_h2 = _make_prop(H2)


def _inv_sqrt_deg(degp_blk):
    deg = jnp.sum(degp_blk, axis=0)
    return lax.rsqrt(jnp.maximum(deg, 1.0))


def _tc1_body(degp_ref, feat_ref, w1_ref, out_ref):
    a = _inv_sqrt_deg(degp_ref[...])
    y = jnp.dot(feat_ref[...], w1_ref[...], preferred_element_type=jnp.float32)
    out_ref[...] = y * a[:, None]


def _tc2_body(p_ref, degp_ref, w2_ref, out_ref):
    a = _inv_sqrt_deg(degp_ref[...])
    s = (p_ref[0, :N] + p_ref[1, :N]) * a[:, None]
    h1 = jnp.maximum(s, 0.0)
    y = jnp.dot(h1, w2_ref[...], preferred_element_type=jnp.float32)
    out_ref[...] = y * a[:, None]


def _tc3_body(p_ref, degp_ref, out_ref):
    a = _inv_sqrt_deg(degp_ref[...])
    out_ref[...] = (p_ref[0, :N] + p_ref[1, :N]) * a[:, None]


def _tc1(degp, feat, W1):
    return pl.pallas_call(
        _tc1_body,
        out_shape=jax.ShapeDtypeStruct((N, H1), jnp.float32),
    )(degp, feat, W1)


def _tc2(p1, degp, W2):
    return pl.pallas_call(
        _tc2_body,
        out_shape=jax.ShapeDtypeStruct((N, H2), jnp.float32),
    )(p1, degp, W2)


def _tc3(p2, degp):
    return pl.pallas_call(
        _tc3_body,
        out_shape=jax.ShapeDtypeStruct((N, H2), jnp.float32),
    )(p2, degp)


def kernel(feat, edge_index, W1, W2):
    if EPT_PAD > EPT:
        pad_src = (
            jnp.arange(NW * (EPT_PAD - EPT), dtype=jnp.int32) * 97 % N
        ).reshape(NW, EPT_PAD - EPT)
        src = jnp.concatenate(
            [edge_index[0].reshape(NW, EPT), pad_src], axis=1
        ).reshape(NW, NCH, CW)
        pad_dst = jnp.broadcast_to(
            N + jnp.arange(EPT_PAD - EPT, dtype=jnp.int32) % (NP - N),
            (NW, EPT_PAD - EPT),
        )
        dst = jnp.concatenate(
            [edge_index[1].reshape(NW, EPT), pad_dst], axis=1
        ).reshape(NW, NCH, CW)
    else:
        src = edge_index[0].reshape(NW, NCH, CW)
        dst = edge_index[1].reshape(NW, NCH, CW)
    dst_flat = edge_index[1]

    degp = _deg_kernel(dst_flat)
    g1 = _tc1(degp, feat, W1)
    p1 = _prop_h1(g1, src, dst)
    g2 = _tc2(p1, degp, W2)
    p2 = _prop_h2(g2, src, dst)
    return _tc3(p2, degp)
